# baseline (speedup 1.0000x reference)
import jax
import jax.numpy as jnp
from jax import lax
from jax.experimental import pallas as pl
from jax.experimental.pallas import tpu as pltpu
from jax.experimental.pallas import tpu_sc as plsc

HID = 128
N_NODES = 10000
N_EDGES = 320000

NC, NS = 2, 16
NW = NC * NS

CHUNK = 80
E_PER_W = N_EDGES // NW
E_ITERS = E_PER_W // CHUNK
X_PAD = NW * 320
X_PER_W = X_PAD // NW
X_ITERS = X_PER_W // CHUNK


def _sc_body(x_idx, A_idx, X_idx, xt, eat, tt, out_x, out_a, out_t,
             idx_v, rows_v, sem):
    wid = lax.axis_index("s") * NC + lax.axis_index("c")

    def lookup(idx_hbm, tab_hbm, out_hbm, base, n_iters):
        def body(i, _):
            off = base + i * CHUNK
            pltpu.sync_copy(idx_hbm.at[pl.ds(off, CHUNK)], idx_v)
            pltpu.async_copy(tab_hbm.at[idx_v], rows_v, sem).wait()
            pltpu.sync_copy(rows_v, out_hbm.at[pl.ds(off, CHUNK)])
            return 0
        lax.fori_loop(0, n_iters, body, 0)

    lookup(A_idx, eat, out_a, wid * E_PER_W, E_ITERS)
    lookup(X_idx, tt, out_t, wid * E_PER_W, E_ITERS)
    lookup(x_idx, xt, out_x, wid * X_PER_W, X_ITERS)


@jax.jit
def _encode(x_idx, A, X, x_table, ea_table, tuple_table):
    mesh = plsc.VectorSubcoreMesh(core_axis_name="c", subcore_axis_name="s",
                                  num_cores=NC, num_subcores=NS)
    run = pl.kernel(
        _sc_body,
        out_type=(
            jax.ShapeDtypeStruct((X_PAD, HID), jnp.float32),
            jax.ShapeDtypeStruct((N_EDGES, HID), jnp.float32),
            jax.ShapeDtypeStruct((N_EDGES, HID), jnp.float32),
        ),
        mesh=mesh,
        scratch_types=[
            pltpu.VMEM((CHUNK,), jnp.int32),
            pltpu.VMEM((CHUNK, HID), jnp.float32),
            pltpu.SemaphoreType.DMA,
        ],
    )
    return run(x_idx, A, X, x_table, ea_table, tuple_table)


def kernel(x, A, X, x_table, ea_table, tuple_table):
    x_idx = jnp.pad(x.reshape(-1), (0, X_PAD - N_NODES))
    out_x, out_a, out_t = _encode(x_idx, A, X, x_table, ea_table, tuple_table)
    return (out_x[:N_NODES], out_a, out_t)


# speedup vs baseline: 11.4606x; 11.4606x over previous
import jax
import jax.numpy as jnp
from jax import lax
from jax.experimental import pallas as pl
from jax.experimental.pallas import tpu as pltpu
from jax.experimental.pallas import tpu_sc as plsc

HID = 128
N_NODES = 10000
N_EDGES = 320000

NC, NS = 2, 16
NW = NC * NS

CHUNK = 80
RING = 5
E_PER_W = N_EDGES // NW
E_CHUNKS = E_PER_W // CHUNK
E_SUPER = E_CHUNKS // RING
X_PAD = NW * 4 * CHUNK
X_PER_W = X_PAD // NW
X_CHUNKS = X_PER_W // CHUNK


def _sc_body(x_idx, A_idx, X_idx, xt, eat, tt, out_x, out_a, out_t,
             ia_v, it_v, ix_v, xt_v, eat_v, tt_v,
             b0, b1, b2, b3, b4, sem_g, sem_s):
    bufs = (b0, b1, b2, b3, b4)
    wid = lax.axis_index("s") * NC + lax.axis_index("c")

    pltpu.sync_copy(A_idx.at[pl.ds(wid * E_PER_W, E_PER_W)], ia_v)
    pltpu.sync_copy(X_idx.at[pl.ds(wid * E_PER_W, E_PER_W)], it_v)
    pltpu.sync_copy(x_idx.at[pl.ds(wid * X_PER_W, X_PER_W)], ix_v)

    @pl.when(lax.axis_index("s") == 0)
    def _stage_tables():
        pltpu.sync_copy(xt, xt_v)
        pltpu.sync_copy(eat, eat_v)
        pltpu.sync_copy(tt, tt_v)

    plsc.subcore_barrier()

    def phase(idx_v, tab_v, out_hbm, base, n_super, ring):
        def super_chunk(i):
            descs = []
            for r in range(ring):
                off = pl.multiple_of((i * ring + r) * CHUNK, CHUNK)
                pltpu.async_copy(
                    tab_v.at[idx_v.at[pl.ds(off, CHUNK)]], bufs[r],
                    sem_g).wait()
                descs.append(pltpu.async_copy(
                    bufs[r],
                    out_hbm.at[pl.ds(pl.multiple_of(base + off, CHUNK),
                                     CHUNK)],
                    sem_s))
            for d in descs:
                d.wait()

        if n_super == 1:
            super_chunk(0)
        else:
            def body(i, _):
                super_chunk(i)
                return 0
            lax.fori_loop(0, n_super, body, 0)

    phase(ia_v, eat_v, out_a, wid * E_PER_W, E_SUPER, RING)
    phase(it_v, tt_v, out_t, wid * E_PER_W, E_SUPER, RING)
    phase(ix_v, xt_v, out_x, wid * X_PER_W, 1, X_CHUNKS)


@jax.jit
def _encode(x_idx, A, X, x_table, ea_table, tuple_table):
    mesh = plsc.VectorSubcoreMesh(core_axis_name="c", subcore_axis_name="s",
                                  num_cores=NC, num_subcores=NS)
    run = pl.kernel(
        _sc_body,
        out_type=(
            jax.ShapeDtypeStruct((X_PAD, HID), jnp.float32),
            jax.ShapeDtypeStruct((N_EDGES, HID), jnp.float32),
            jax.ShapeDtypeStruct((N_EDGES, HID), jnp.float32),
        ),
        mesh=mesh,
        scratch_types=[
            pltpu.VMEM((E_PER_W,), jnp.int32),
            pltpu.VMEM((E_PER_W,), jnp.int32),
            pltpu.VMEM((X_PER_W,), jnp.int32),
            pltpu.VMEM_SHARED((32, HID), jnp.float32),
            pltpu.VMEM_SHARED((16, HID), jnp.float32),
            pltpu.VMEM_SHARED((16, HID), jnp.float32),
        ] + [pltpu.VMEM((CHUNK, HID), jnp.float32) for _ in range(RING)]
          + [pltpu.SemaphoreType.DMA, pltpu.SemaphoreType.DMA],
    )
    return run(x_idx, A, X, x_table, ea_table, tuple_table)


def kernel(x, A, X, x_table, ea_table, tuple_table):
    x_idx = jnp.pad(x.reshape(-1), (0, X_PAD - N_NODES))
    out_x, out_a, out_t = _encode(x_idx, A, X, x_table, ea_table, tuple_table)
    return (out_x[:N_NODES], out_a, out_t)


# speedup vs baseline: 13.3735x; 1.1669x over previous
import jax
import jax.numpy as jnp
from jax import lax
from jax.experimental import pallas as pl
from jax.experimental.pallas import tpu as pltpu
from jax.experimental.pallas import tpu_sc as plsc

HID = 128
N_NODES = 10000
N_EDGES = 320000

NC, NS = 2, 16
NW = NC * NS

CHUNK = 80
RING = 5
E_PER_W = N_EDGES // NW
E_CHUNKS = E_PER_W // CHUNK
E_SUPER = E_CHUNKS // RING
X_PAD = NW * 4 * CHUNK
X_PER_W = X_PAD // NW
X_CHUNKS = X_PER_W // CHUNK

TC_BLK = 2560
TC_GRID = N_EDGES // TC_BLK


def _sc_body(x_idx, X_idx, xt, tt, out_x, out_t,
             it_v, ix_v, xt_v, tt_v,
             b0, b1, b2, b3, b4, sem_g, sem_s):
    bufs = (b0, b1, b2, b3, b4)
    wid = lax.axis_index("s") * NC + lax.axis_index("c")

    pltpu.sync_copy(X_idx.at[pl.ds(wid * E_PER_W, E_PER_W)], it_v)
    pltpu.sync_copy(x_idx.at[pl.ds(wid * X_PER_W, X_PER_W)], ix_v)

    @pl.when(lax.axis_index("s") == 0)
    def _stage_tables():
        pltpu.sync_copy(xt, xt_v)
        pltpu.sync_copy(tt, tt_v)

    plsc.subcore_barrier()

    def phase(idx_v, tab_v, out_hbm, base, n_super, ring):
        def super_chunk(i):
            descs = []
            for r in range(ring):
                off = pl.multiple_of((i * ring + r) * CHUNK, CHUNK)
                pltpu.async_copy(
                    tab_v.at[idx_v.at[pl.ds(off, CHUNK)]], bufs[r],
                    sem_g).wait()
                descs.append(pltpu.async_copy(
                    bufs[r],
                    out_hbm.at[pl.ds(pl.multiple_of(base + off, CHUNK),
                                     CHUNK)],
                    sem_s))
            for d in descs:
                d.wait()

        if n_super == 1:
            super_chunk(0)
        else:
            def body(i, _):
                super_chunk(i)
                return 0
            lax.fori_loop(0, n_super, body, 0)

    phase(it_v, tt_v, out_t, wid * E_PER_W, E_SUPER, RING)
    phase(ix_v, xt_v, out_x, wid * X_PER_W, 1, X_CHUNKS)


def _tc_body(idx_ref, tab_ref, out_ref):
    idx = idx_ref[0, 0, :]
    onehot = (idx[:, None] ==
              lax.broadcasted_iota(jnp.int32, (TC_BLK, 16), 1)
              ).astype(jnp.float32)
    out_ref[...] = jnp.dot(onehot, tab_ref[...],
                           preferred_element_type=jnp.float32)


@jax.jit
def _encode(x_idx, A, X, x_table, ea_table, tuple_table):
    mesh = plsc.VectorSubcoreMesh(core_axis_name="c", subcore_axis_name="s",
                                  num_cores=NC, num_subcores=NS)
    sc_run = pl.kernel(
        _sc_body,
        out_type=(
            jax.ShapeDtypeStruct((X_PAD, HID), jnp.float32),
            jax.ShapeDtypeStruct((N_EDGES, HID), jnp.float32),
        ),
        mesh=mesh,
        scratch_types=[
            pltpu.VMEM((E_PER_W,), jnp.int32),
            pltpu.VMEM((X_PER_W,), jnp.int32),
            pltpu.VMEM_SHARED((32, HID), jnp.float32),
            pltpu.VMEM_SHARED((16, HID), jnp.float32),
        ] + [pltpu.VMEM((CHUNK, HID), jnp.float32) for _ in range(RING)]
          + [pltpu.SemaphoreType.DMA, pltpu.SemaphoreType.DMA],
    )
    out_x, out_t = sc_run(x_idx, X, x_table, tuple_table)

    out_a = pl.pallas_call(
        _tc_body,
        grid=(TC_GRID,),
        in_specs=[
            pl.BlockSpec((1, 1, TC_BLK), lambda i: (i, 0, 0)),
            pl.BlockSpec((16, HID), lambda i: (0, 0)),
        ],
        out_specs=pl.BlockSpec((TC_BLK, HID), lambda i: (i, 0)),
        out_shape=jax.ShapeDtypeStruct((N_EDGES, HID), jnp.float32),
    )(A.reshape(TC_GRID, 1, TC_BLK), ea_table)

    return out_x, out_a, out_t


def kernel(x, A, X, x_table, ea_table, tuple_table):
    x_idx = jnp.pad(x.reshape(-1), (0, X_PAD - N_NODES))
    out_x, out_a, out_t = _encode(x_idx, A, X, x_table, ea_table, tuple_table)
    return (out_x[:N_NODES], out_a, out_t)


# speedup vs baseline: 16.1071x; 1.2044x over previous
import jax
import jax.numpy as jnp
from jax import lax
from jax.experimental import pallas as pl
from jax.experimental.pallas import tpu as pltpu
from jax.experimental.pallas import tpu_sc as plsc

HID = 128
N_NODES = 10000
N_EDGES = 320000

NC, NS = 2, 16
NW = NC * NS

CHUNK = 80
RING = 5
E_PER_W = N_EDGES // NW
E_CHUNKS = E_PER_W // CHUNK
E_SUPER = E_CHUNKS // RING
X_PAD = NW * 4 * CHUNK
X_PER_W = X_PAD // NW
X_CHUNKS = X_PER_W // CHUNK

TC_BLK = 3200
TC_GRID = N_EDGES // (2 * TC_BLK)


def _sc_body(x_idx, X_idx, xt, tt, out_x, out_t,
             it_v, ix_v, xt_v, tt_v,
             b0, b1, b2, b3, b4, sem_g, sem_s):
    bufs = (b0, b1, b2, b3, b4)
    wid = lax.axis_index("s") * NC + lax.axis_index("c")

    pltpu.sync_copy(X_idx.at[pl.ds(wid * E_PER_W, E_PER_W)], it_v)
    pltpu.sync_copy(x_idx.at[pl.ds(wid * X_PER_W, X_PER_W)], ix_v)

    @pl.when(lax.axis_index("s") == 0)
    def _stage_tables():
        pltpu.sync_copy(xt, xt_v)
        pltpu.sync_copy(tt, tt_v)

    plsc.subcore_barrier()

    def phase(idx_v, tab_v, out_hbm, base, n_super, ring):
        def super_chunk(i):
            descs = []
            for r in range(ring):
                off = pl.multiple_of((i * ring + r) * CHUNK, CHUNK)
                pltpu.async_copy(
                    tab_v.at[idx_v.at[pl.ds(off, CHUNK)]], bufs[r],
                    sem_g).wait()
                descs.append(pltpu.async_copy(
                    bufs[r],
                    out_hbm.at[pl.ds(pl.multiple_of(base + off, CHUNK),
                                     CHUNK)],
                    sem_s))
            for d in descs:
                d.wait()

        if n_super == 1:
            super_chunk(0)
        else:
            def body(i, _):
                super_chunk(i)
                return 0
            lax.fori_loop(0, n_super, body, 0)

    phase(it_v, tt_v, out_t, wid * E_PER_W, E_SUPER, RING)
    phase(ix_v, xt_v, out_x, wid * X_PER_W, 1, X_CHUNKS)


def _tc_body(idx_ref, tab2_ref, out_ref):
    idx = idx_ref[0, 0, :]
    k = lax.broadcasted_iota(jnp.int32, (TC_BLK, 32), 1)
    idxsel = jnp.where(k < 16, idx[:TC_BLK, None], idx[TC_BLK:, None])
    oh = (idxsel == (k & 15)).astype(jnp.float32)
    res = jnp.dot(oh, tab2_ref[...], preferred_element_type=jnp.float32)
    out_ref[:TC_BLK, :] = res[:, :HID]
    out_ref[TC_BLK:, :] = res[:, HID:]


@jax.jit
def _encode(x_idx, A, X, x_table, ea_table, tuple_table):
    mesh = plsc.VectorSubcoreMesh(core_axis_name="c", subcore_axis_name="s",
                                  num_cores=NC, num_subcores=NS)
    sc_run = pl.kernel(
        _sc_body,
        out_type=(
            jax.ShapeDtypeStruct((X_PAD, HID), jnp.float32),
            jax.ShapeDtypeStruct((N_EDGES, HID), jnp.float32),
        ),
        mesh=mesh,
        scratch_types=[
            pltpu.VMEM((E_PER_W,), jnp.int32),
            pltpu.VMEM((X_PER_W,), jnp.int32),
            pltpu.VMEM_SHARED((32, HID), jnp.float32),
            pltpu.VMEM_SHARED((16, HID), jnp.float32),
        ] + [pltpu.VMEM((CHUNK, HID), jnp.float32) for _ in range(RING)]
          + [pltpu.SemaphoreType.DMA, pltpu.SemaphoreType.DMA],
    )
    out_x, out_t = sc_run(x_idx, X, x_table, tuple_table)

    tab2 = jnp.zeros((32, 2 * HID), jnp.float32)
    tab2 = tab2.at[:16, :HID].set(ea_table).at[16:, HID:].set(ea_table)
    out_a = pl.pallas_call(
        _tc_body,
        grid=(TC_GRID,),
        in_specs=[
            pl.BlockSpec((1, 1, 2 * TC_BLK), lambda i: (i, 0, 0)),
            pl.BlockSpec((32, 2 * HID), lambda i: (0, 0)),
        ],
        out_specs=pl.BlockSpec((2 * TC_BLK, HID), lambda i: (i, 0)),
        out_shape=jax.ShapeDtypeStruct((N_EDGES, HID), jnp.float32),
    )(A.reshape(TC_GRID, 1, 2 * TC_BLK), tab2)

    return out_x, out_a, out_t


def kernel(x, A, X, x_table, ea_table, tuple_table):
    x_idx = jnp.pad(x.reshape(-1), (0, X_PAD - N_NODES))
    out_x, out_a, out_t = _encode(x_idx, A, X, x_table, ea_table, tuple_table)
    return (out_x[:N_NODES], out_a, out_t)


# speedup vs baseline: 16.6341x; 1.0327x over previous
import jax
import jax.numpy as jnp
from jax import lax
from jax.experimental import pallas as pl
from jax.experimental.pallas import tpu as pltpu
from jax.experimental.pallas import tpu_sc as plsc

HID = 128
N_NODES = 10000
N_EDGES = 320000

NC, NS = 2, 16
NW = NC * NS

CHUNK = 80
RING = 5
E_PER_W = N_EDGES // NW
E_CHUNKS = E_PER_W // CHUNK
E_SUPER = E_CHUNKS // RING

X_PER_W = 312
X_SUB = 104
X_TAIL = N_NODES - NW * X_PER_W

TC_BLK = 3200
TC_GRID = N_EDGES // (2 * TC_BLK)


def _sc_body(x_idx, X_idx, xt, tt, out_x, out_t,
             it_v, ix_v, xt_v, tt_v,
             b0, b1, b2, b3, b4, xb0, xb1, xb2, xbt, sem_g, sem_s):
    bufs = (b0, b1, b2, b3, b4)
    xbufs = (xb0, xb1, xb2)
    wid = lax.axis_index("s") * NC + lax.axis_index("c")

    pltpu.sync_copy(X_idx.at[pl.ds(wid * E_PER_W, E_PER_W)], it_v)
    pltpu.sync_copy(x_idx.at[pl.ds(wid * X_PER_W, X_PER_W)],
                    ix_v.at[pl.ds(0, X_PER_W)])

    @pl.when(lax.axis_index("s") == 0)
    def _stage_tables():
        pltpu.sync_copy(xt, xt_v)
        pltpu.sync_copy(tt, tt_v)

    plsc.subcore_barrier()

    def super_chunk(i):
        descs = []
        for r in range(RING):
            off = pl.multiple_of((i * RING + r) * CHUNK, CHUNK)
            pltpu.async_copy(
                tt_v.at[it_v.at[pl.ds(off, CHUNK)]], bufs[r], sem_g).wait()
            descs.append(pltpu.async_copy(
                bufs[r],
                out_t.at[pl.ds(pl.multiple_of(wid * E_PER_W + off, CHUNK),
                               CHUNK)],
                sem_s))
        for d in descs:
            d.wait()

    def body(i, _):
        super_chunk(i)
        return 0
    lax.fori_loop(0, E_SUPER, body, 0)

    xdescs = []
    for r in range(3):
        off = pl.multiple_of(r * X_SUB, 8)
        pltpu.async_copy(
            xt_v.at[ix_v.at[pl.ds(off, X_SUB)]], xbufs[r], sem_g).wait()
        xdescs.append(pltpu.async_copy(
            xbufs[r],
            out_x.at[pl.ds(pl.multiple_of(wid * X_PER_W, 8) + off, X_SUB)],
            sem_s))

    @pl.when(wid == NW - 1)
    def _tail():
        pltpu.sync_copy(x_idx.at[pl.ds(NW * X_PER_W, X_TAIL)],
                        ix_v.at[pl.ds(X_PER_W, X_TAIL)])
        pltpu.async_copy(
            xt_v.at[ix_v.at[pl.ds(X_PER_W, X_TAIL)]], xbt, sem_g).wait()
        pltpu.async_copy(
            xbt, out_x.at[pl.ds(NW * X_PER_W, X_TAIL)], sem_s).wait()

    for d in xdescs:
        d.wait()


def _tc_body(idx_ref, tab2_ref, out_ref):
    idx = idx_ref[0, 0, :]
    k = lax.broadcasted_iota(jnp.int32, (TC_BLK, 32), 1)
    idxsel = jnp.where(k < 16, idx[:TC_BLK, None], idx[TC_BLK:, None])
    oh = (idxsel == (k & 15)).astype(jnp.float32)
    res = jnp.dot(oh, tab2_ref[...], preferred_element_type=jnp.float32)
    out_ref[:TC_BLK, :] = res[:, :HID]
    out_ref[TC_BLK:, :] = res[:, HID:]


@jax.jit
def _encode(x_idx, A, X, x_table, ea_table, tuple_table):
    mesh = plsc.VectorSubcoreMesh(core_axis_name="c", subcore_axis_name="s",
                                  num_cores=NC, num_subcores=NS)
    sc_run = pl.kernel(
        _sc_body,
        out_type=(
            jax.ShapeDtypeStruct((N_NODES, HID), jnp.float32),
            jax.ShapeDtypeStruct((N_EDGES, HID), jnp.float32),
        ),
        mesh=mesh,
        scratch_types=[
            pltpu.VMEM((E_PER_W,), jnp.int32),
            pltpu.VMEM((X_PER_W + X_TAIL,), jnp.int32),
            pltpu.VMEM_SHARED((32, HID), jnp.float32),
            pltpu.VMEM_SHARED((16, HID), jnp.float32),
        ] + [pltpu.VMEM((CHUNK, HID), jnp.float32) for _ in range(RING)]
          + [pltpu.VMEM((X_SUB, HID), jnp.float32) for _ in range(3)]
          + [pltpu.VMEM((X_TAIL, HID), jnp.float32),
             pltpu.SemaphoreType.DMA, pltpu.SemaphoreType.DMA],
    )
    out_x, out_t = sc_run(x_idx, X, x_table, tuple_table)

    tab2 = jnp.zeros((32, 2 * HID), jnp.float32)
    tab2 = tab2.at[:16, :HID].set(ea_table).at[16:, HID:].set(ea_table)
    out_a = pl.pallas_call(
        _tc_body,
        grid=(TC_GRID,),
        in_specs=[
            pl.BlockSpec((1, 1, 2 * TC_BLK), lambda i: (i, 0, 0)),
            pl.BlockSpec((32, 2 * HID), lambda i: (0, 0)),
        ],
        out_specs=pl.BlockSpec((2 * TC_BLK, HID), lambda i: (i, 0)),
        out_shape=jax.ShapeDtypeStruct((N_EDGES, HID), jnp.float32),
    )(A.reshape(TC_GRID, 1, 2 * TC_BLK), tab2)

    return out_x, out_a, out_t


def kernel(x, A, X, x_table, ea_table, tuple_table):
    return _encode(x.reshape(-1), A, X, x_table, ea_table, tuple_table)


# speedup vs baseline: 17.4593x; 1.0496x over previous
import jax
import jax.numpy as jnp
from jax import lax
from jax.experimental import pallas as pl
from jax.experimental.pallas import tpu as pltpu
from jax.experimental.pallas import tpu_sc as plsc

HID = 128
N_NODES = 10000
N_EDGES = 320000

NC, NS = 2, 16
NW = NC * NS

CHUNK = 80
RING = 5
E_PER_W = N_EDGES // NW
E_CHUNKS = E_PER_W // CHUNK
E_SUPER = E_CHUNKS // RING

X_PER_W = 312
X_SUB = 104
X_TAIL = N_NODES - NW * X_PER_W

TC_BLK = 6400
TC_GRID = N_EDGES // (2 * TC_BLK)


def _sc_body(x_idx, X_idx, xt, tt, out_x, out_t,
             it_v, ix_v, xt_v, tt_v,
             b0, b1, b2, b3, b4, xb0, xb1, xb2, xbt, sem_g, sem_s):
    bufs = (b0, b1, b2, b3, b4)
    xbufs = (xb0, xb1, xb2)
    wid = lax.axis_index("s") * NC + lax.axis_index("c")

    pltpu.sync_copy(X_idx.at[pl.ds(wid * E_PER_W, E_PER_W)], it_v)
    pltpu.sync_copy(x_idx.at[pl.ds(wid * X_PER_W, X_PER_W)],
                    ix_v.at[pl.ds(0, X_PER_W)])

    @pl.when(lax.axis_index("s") == 0)
    def _stage_tables():
        pltpu.sync_copy(xt, xt_v)
        pltpu.sync_copy(tt, tt_v)

    plsc.subcore_barrier()

    def super_chunk(i):
        descs = []
        for r in range(RING):
            off = pl.multiple_of((i * RING + r) * CHUNK, CHUNK)
            pltpu.async_copy(
                tt_v.at[it_v.at[pl.ds(off, CHUNK)]], bufs[r], sem_g).wait()
            descs.append(pltpu.async_copy(
                bufs[r],
                out_t.at[pl.ds(pl.multiple_of(wid * E_PER_W + off, CHUNK),
                               CHUNK)],
                sem_s))
        for d in descs:
            d.wait()

    def body(i, _):
        super_chunk(i)
        return 0
    lax.fori_loop(0, E_SUPER, body, 0)

    xdescs = []
    for r in range(3):
        off = pl.multiple_of(r * X_SUB, 8)
        pltpu.async_copy(
            xt_v.at[ix_v.at[pl.ds(off, X_SUB)]], xbufs[r], sem_g).wait()
        xdescs.append(pltpu.async_copy(
            xbufs[r],
            out_x.at[pl.ds(pl.multiple_of(wid * X_PER_W, 8) + off, X_SUB)],
            sem_s))

    @pl.when(wid == NW - 1)
    def _tail():
        pltpu.sync_copy(x_idx.at[pl.ds(NW * X_PER_W, X_TAIL)],
                        ix_v.at[pl.ds(X_PER_W, X_TAIL)])
        pltpu.async_copy(
            xt_v.at[ix_v.at[pl.ds(X_PER_W, X_TAIL)]], xbt, sem_g).wait()
        pltpu.async_copy(
            xbt, out_x.at[pl.ds(NW * X_PER_W, X_TAIL)], sem_s).wait()

    for d in xdescs:
        d.wait()


def _tc_body(idx_ref, tab2_ref, out_ref):
    i = pl.program_id(0)
    idx = idx_ref[pl.ds(i * 2 * TC_BLK, 2 * TC_BLK)]
    k = lax.broadcasted_iota(jnp.int32, (TC_BLK, 32), 1)
    idxsel = jnp.where(k < 16, idx[:TC_BLK, None], idx[TC_BLK:, None])
    oh = (idxsel == (k & 15)).astype(jnp.float32)
    res = jnp.dot(oh, tab2_ref[...], preferred_element_type=jnp.float32)
    out_ref[:TC_BLK, :] = res[:, :HID]
    out_ref[TC_BLK:, :] = res[:, HID:]


@jax.jit
def _encode(x_idx, A, X, x_table, ea_table, tuple_table):
    mesh = plsc.VectorSubcoreMesh(core_axis_name="c", subcore_axis_name="s",
                                  num_cores=NC, num_subcores=NS)
    sc_run = pl.kernel(
        _sc_body,
        out_type=(
            jax.ShapeDtypeStruct((N_NODES, HID), jnp.float32),
            jax.ShapeDtypeStruct((N_EDGES, HID), jnp.float32),
        ),
        mesh=mesh,
        scratch_types=[
            pltpu.VMEM((E_PER_W,), jnp.int32),
            pltpu.VMEM((X_PER_W + X_TAIL,), jnp.int32),
            pltpu.VMEM_SHARED((32, HID), jnp.float32),
            pltpu.VMEM_SHARED((16, HID), jnp.float32),
        ] + [pltpu.VMEM((CHUNK, HID), jnp.float32) for _ in range(RING)]
          + [pltpu.VMEM((X_SUB, HID), jnp.float32) for _ in range(3)]
          + [pltpu.VMEM((X_TAIL, HID), jnp.float32),
             pltpu.SemaphoreType.DMA, pltpu.SemaphoreType.DMA],
    )
    out_x, out_t = sc_run(x_idx, X, x_table, tuple_table)

    tab2 = jnp.zeros((32, 2 * HID), jnp.float32)
    tab2 = tab2.at[:16, :HID].set(ea_table).at[16:, HID:].set(ea_table)
    out_a = pl.pallas_call(
        _tc_body,
        grid=(TC_GRID,),
        in_specs=[
            pl.BlockSpec((N_EDGES,), lambda i: (0,)),
            pl.BlockSpec((32, 2 * HID), lambda i: (0, 0)),
        ],
        out_specs=pl.BlockSpec((2 * TC_BLK, HID), lambda i: (i, 0)),
        out_shape=jax.ShapeDtypeStruct((N_EDGES, HID), jnp.float32),
    )(A, tab2)

    return out_x, out_a, out_t


def kernel(x, A, X, x_table, ea_table, tuple_table):
    return _encode(x.reshape(-1), A, X, x_table, ea_table, tuple_table)


# speedup vs baseline: 17.6756x; 1.0124x over previous
import jax
import jax.numpy as jnp
from jax import lax
from jax.experimental import pallas as pl
from jax.experimental.pallas import tpu as pltpu
from jax.experimental.pallas import tpu_sc as plsc

HID = 128
N_NODES = 10000
N_EDGES = 320000

NC, NS = 2, 16
NW = NC * NS

CHUNK = 80
RING = 5
E_PER_W = N_EDGES // NW
E_CHUNKS = E_PER_W // CHUNK
E_SUPER = E_CHUNKS // RING

X_PER_W = 312
X_SUB = 104
X_TAIL = N_NODES - NW * X_PER_W

TC_BLK = 6400
TC_GRID = N_EDGES // (2 * TC_BLK)


def _sc_body(x_idx, X_idx, xt, tt, out_x, out_t,
             it_v, ix_v, xt_v, tt_v,
             b0, b1, b2, b3, b4, xb0, xb1, xb2, xbt, sem_g, sem_s):
    bufs = (b0, b1, b2, b3, b4)
    xbufs = (xb0, xb1, xb2)
    wid = lax.axis_index("s") * NC + lax.axis_index("c")

    pltpu.sync_copy(X_idx.at[pl.ds(wid * E_PER_W, E_PER_W)], it_v)
    pltpu.sync_copy(x_idx.at[pl.ds(wid * X_PER_W, X_PER_W)],
                    ix_v.at[pl.ds(0, X_PER_W)])

    @pl.when(lax.axis_index("s") == 0)
    def _stage_tables():
        pltpu.sync_copy(xt, xt_v)
        pltpu.sync_copy(tt, tt_v)

    plsc.subcore_barrier()

    def super_chunk(i):
        descs = []
        for r in range(RING):
            off = pl.multiple_of((i * RING + r) * CHUNK, CHUNK)
            pltpu.async_copy(
                tt_v.at[it_v.at[pl.ds(off, CHUNK)]], bufs[r], sem_g).wait()
            descs.append(pltpu.async_copy(
                bufs[r],
                out_t.at[pl.ds(pl.multiple_of(wid * E_PER_W + off, CHUNK),
                               CHUNK)],
                sem_s))
        for d in descs:
            d.wait()

    def body(i, _):
        super_chunk(i)
        return 0
    lax.fori_loop(0, E_SUPER, body, 0)

    xdescs = []
    for r in range(3):
        off = pl.multiple_of(r * X_SUB, 8)
        pltpu.async_copy(
            xt_v.at[ix_v.at[pl.ds(off, X_SUB)]], xbufs[r], sem_g).wait()
        xdescs.append(pltpu.async_copy(
            xbufs[r],
            out_x.at[pl.ds(pl.multiple_of(wid * X_PER_W, 8) + off, X_SUB)],
            sem_s))

    @pl.when(wid == NW - 1)
    def _tail():
        pltpu.sync_copy(x_idx.at[pl.ds(NW * X_PER_W, X_TAIL)],
                        ix_v.at[pl.ds(X_PER_W, X_TAIL)])
        pltpu.async_copy(
            xt_v.at[ix_v.at[pl.ds(X_PER_W, X_TAIL)]], xbt, sem_g).wait()
        pltpu.async_copy(
            xbt, out_x.at[pl.ds(NW * X_PER_W, X_TAIL)], sem_s).wait()

    for d in xdescs:
        d.wait()


def _tc_body(idx_ref, tab_ref, out_ref, tab2_s):
    i = pl.program_id(0)

    @pl.when(i == 0)
    def _build_tab2():
        tab2_s[...] = jnp.zeros((32, 2 * HID), jnp.float32)
        tab2_s[:16, :HID] = tab_ref[...]
        tab2_s[16:, HID:] = tab_ref[...]

    idx = idx_ref[pl.ds(i * 2 * TC_BLK, 2 * TC_BLK)]
    k = lax.broadcasted_iota(jnp.int32, (TC_BLK, 32), 1)
    idxsel = jnp.where(k < 16, idx[:TC_BLK, None], idx[TC_BLK:, None])
    oh = (idxsel == (k & 15)).astype(jnp.float32)
    res = jnp.dot(oh, tab2_s[...], preferred_element_type=jnp.float32)
    out_ref[:TC_BLK, :] = res[:, :HID]
    out_ref[TC_BLK:, :] = res[:, HID:]


@jax.jit
def _encode(x_idx, A, X, x_table, ea_table, tuple_table):
    mesh = plsc.VectorSubcoreMesh(core_axis_name="c", subcore_axis_name="s",
                                  num_cores=NC, num_subcores=NS)
    sc_run = pl.kernel(
        _sc_body,
        out_type=(
            jax.ShapeDtypeStruct((N_NODES, HID), jnp.float32),
            jax.ShapeDtypeStruct((N_EDGES, HID), jnp.float32),
        ),
        mesh=mesh,
        scratch_types=[
            pltpu.VMEM((E_PER_W,), jnp.int32),
            pltpu.VMEM((X_PER_W + X_TAIL,), jnp.int32),
            pltpu.VMEM_SHARED((32, HID), jnp.float32),
            pltpu.VMEM_SHARED((16, HID), jnp.float32),
        ] + [pltpu.VMEM((CHUNK, HID), jnp.float32) for _ in range(RING)]
          + [pltpu.VMEM((X_SUB, HID), jnp.float32) for _ in range(3)]
          + [pltpu.VMEM((X_TAIL, HID), jnp.float32),
             pltpu.SemaphoreType.DMA, pltpu.SemaphoreType.DMA],
    )
    out_x, out_t = sc_run(x_idx, X, x_table, tuple_table)

    out_a = pl.pallas_call(
        _tc_body,
        grid=(TC_GRID,),
        in_specs=[
            pl.BlockSpec((N_EDGES,), lambda i: (0,)),
            pl.BlockSpec((16, HID), lambda i: (0, 0)),
        ],
        out_specs=pl.BlockSpec((2 * TC_BLK, HID), lambda i: (i, 0)),
        out_shape=jax.ShapeDtypeStruct((N_EDGES, HID), jnp.float32),
        scratch_shapes=[pltpu.VMEM((32, 2 * HID), jnp.float32)],
    )(A, ea_table)

    return out_x, out_a, out_t


def kernel(x, A, X, x_table, ea_table, tuple_table):
    return _encode(x.reshape(-1), A, X, x_table, ea_table, tuple_table)
